# baseline (speedup 1.0000x reference)
import jax
import jax.numpy as jnp
from jax.experimental import pallas as pl
from jax.experimental.pallas import tpu as pltpu

_B = 8
_NR = 5000
_NRP = 5120
_NCH = _NRP // 128
_NG = 100
_NGP = 104
_FG = 64
_REG_FG_THRESH = 0.55
_CLS_FG_THRESH = 0.75
_CLS_BG_THRESH = 0.25

_HIGHEST = jax.lax.Precision.HIGHEST
_DNL = (((1,), (0,)), ((), ()))
_DNT = (((1,), (1,)), ((), ()))


def _roll(x, s):
    s = s % x.shape[1]
    if s == 0:
        return x
    return jnp.concatenate([x[:, -s:], x[:, :-s]], axis=1)


def _iou_block(axM, axm, ayM, aym, azM, azm, va, gb):
    xM, xm = gb[:, 0:1], gb[:, 1:2]
    yM, ym = gb[:, 2:3], gb[:, 3:4]
    zM, zm = gb[:, 4:5], gb[:, 5:6]
    vb = gb[:, 6:7]
    ox = jnp.maximum(jnp.minimum(axM, xM) - jnp.maximum(axm, xm), 0.0)
    oy = jnp.maximum(jnp.minimum(ayM, yM) - jnp.maximum(aym, ym), 0.0)
    oz = jnp.maximum(jnp.minimum(azM, zM) - jnp.maximum(azm, zm), 0.0)
    inter = ox * oy * oz
    return inter / jnp.maximum((va + vb) - inter, 1e-6)


def _tc_body(data_ref, gtb_ref, gtt_ref, g_ref, gtout_ref, reg_ref, cls_ref):
    f32 = jnp.float32
    i32 = jnp.int32

    mo_imgs = []
    for b in range(_B):
        d = data_ref[b]
        gb = gtb_ref[b]
        rows = []
        for j in range(_NCH):
            lo = j * 128
            iou = _iou_block(
                d[9:10, lo:lo + 128], d[10:11, lo:lo + 128],
                d[11:12, lo:lo + 128], d[12:13, lo:lo + 128],
                d[13:14, lo:lo + 128], d[14:15, lo:lo + 128],
                d[15:16, lo:lo + 128], gb)
            rows.append(jnp.max(iou, axis=0, keepdims=True))
        mo_imgs.append(jnp.concatenate(rows, axis=0))
    mo3 = jnp.stack(mo_imgs)

    r3 = jax.lax.broadcasted_iota(i32, (_B, _NCH, 128), 1)
    c3 = jax.lax.broadcasted_iota(i32, (_B, _NCH, 128), 2)
    valid3 = (r3 * 128 + c3) < _NR
    bits3 = jax.lax.bitcast_convert_type(mo3, i32)
    bitsf = jnp.where(valid3, bits3, -1)
    bitsb = jnp.where(valid3, bits3, jnp.int32(0x7FFFFFFF))

    def sbody(i, carry):
        tf_, tb_ = carry
        bit = jax.lax.shift_left(jnp.int32(1), jnp.int32(30) - i)
        candf = tf_ | bit
        cf = jnp.sum((bitsf >= candf).astype(i32), axis=(1, 2), keepdims=True)
        tf_ = jnp.where(cf >= _FG, candf, tf_)
        candb = tb_ & jnp.bitwise_not(bit)
        cb = jnp.sum((bitsb <= candb).astype(i32), axis=(1, 2), keepdims=True)
        tb_ = jnp.where(cb >= _FG, candb, tb_)
        return tf_, tb_

    tfb_i, tbb_i = jax.lax.fori_loop(
        0, 31, sbody,
        (jnp.zeros((_B, 1, 1), i32), jnp.full((_B, 1, 1), 0x7FFFFFFF, i32)))
    tf3 = jax.lax.bitcast_convert_type(tfb_i, f32)
    tb3 = jax.lax.bitcast_convert_type(tbb_i, f32)

    io0 = jax.lax.broadcasted_iota(i32, (128, 128), 0)
    io1 = jax.lax.broadcasted_iota(i32, (128, 128), 1)
    tl128 = (io0 <= io1).astype(f32)
    i128 = (io0 == io1).astype(f32)
    q0 = jax.lax.broadcasted_iota(i32, (4 * _NCH, 4 * _NCH), 0)
    q1 = jax.lax.broadcasted_iota(i32, (4 * _NCH, 4 * _NCH), 1)
    same = (((q0 < 40) & (q1 < 40))
            | ((q0 >= 40) & (q0 < 80) & (q1 >= 40) & (q1 < 80))
            | ((q0 >= 80) & (q0 < 120) & (q1 >= 80) & (q1 < 120))
            | ((q0 >= 120) & (q1 >= 120)))
    sblk = (same & (q1 < q0)).astype(f32)
    r2d = jax.lax.broadcasted_iota(i32, (_NCH, 128), 0)
    c2d = jax.lax.broadcasted_iota(i32, (_NCH, 128), 1)
    valid2 = (r2d * 128 + c2d) < _NR
    lane_f = jax.lax.broadcasted_iota(i32, (1, 128), 1).astype(f32)
    giota = jax.lax.broadcasted_iota(i32, (_NGP, 1), 0)
    giota_f = giota.astype(f32)

    moc_list = []
    gac_list = []
    g9_list = []
    for b in range(_B):
        mob = mo_imgs[b]
        tfv = tf3[b]
        tbv = tb3[b]
        betf = valid2 & (mob > tfv)
        tief = valid2 & (mob == tfv)
        betb = valid2 & (mob < tbv)
        tieb = valid2 & (mob == tbv)
        needf = jnp.float32(_FG) - jnp.sum(betf.astype(f32), keepdims=True)
        needb = jnp.float32(_FG) - jnp.sum(betb.astype(f32), keepdims=True)
        mstk = jnp.concatenate([betf, tief, betb, tieb],
                               axis=0).astype(f32)
        inc = jax.lax.dot_general(mstk, tl128, _DNL,
                                  preferred_element_type=f32)
        rowoff = jax.lax.dot_general(sblk, inc[:, 127:128], _DNL,
                                     preferred_element_type=f32)
        pstk = (inc - mstk) + rowoff
        pbf, ptf = pstk[0:40], pstk[40:80]
        pbb, ptb = pstk[80:120], pstk[120:160]
        self_f = betf | (tief & (ptf < needf))
        sf = jnp.where(self_f, pbf + jnp.minimum(ptf, needf), 200.0)
        self_b = betb | (tieb & (ptb < needb))
        sb_ = jnp.where(self_b, 64.0 + pbb + jnp.minimum(ptb, needb), 200.0)
        s2 = jnp.concatenate([sf, sb_], axis=0)
        scol = jax.lax.dot_general(i128, s2, _DNT,
                                   preferred_element_type=f32)
        blocks = []
        for r in range(_NCH):
            blk = ((scol[:, r:r + 1] == lane_f)
                   | (scol[:, _NCH + r:_NCH + r + 1] == lane_f))
            blocks.append(blk.astype(f32))
        sel = jnp.concatenate(blocks, axis=0)
        g9 = jax.lax.dot_general(data_ref[b][0:9], sel, _DNL,
                                 precision=_HIGHEST,
                                 preferred_element_type=f32)
        x, y, z = g9[0:1], g9[1:2], g9[2:3]
        dx, dy, dz = g9[3:4], g9[4:5], g9[5:6]
        iou2 = _iou_block(x + dx * 0.5, x - dx * 0.5,
                          y + dy * 0.5, y - dy * 0.5,
                          z + dz * 0.5, z - dz * 0.5,
                          (dx * dy) * dz, gtb_ref[b])
        moc = jnp.max(iou2, axis=0, keepdims=True)
        gac = jnp.min(jnp.where(iou2 == moc, giota, _NGP),
                      axis=0, keepdims=True)
        moc_list.append(moc)
        gac_list.append(gac.astype(f32))
        g9_list.append(g9)

    V = jnp.concatenate(moc_list, axis=0)
    bitsV = jax.lax.bitcast_convert_type(V, i32)
    lane8 = jax.lax.broadcasted_iota(i32, (_B, 128), 1)
    half_fg = lane8 < _FG
    P = jnp.where(half_fg, -bitsV, bitsV + jnp.int32(1 << 30))
    S = lane8
    k = 2
    while k <= 128:
        d = k // 2
        while d >= 1:
            upper = (lane8 & d) != 0
            pP = jnp.where(upper, _roll(P, d), _roll(P, -d))
            pS = jnp.where(upper, _roll(S, d), _roll(S, -d))
            self_less = (P < pP) | ((P == pP) & (S < pS))
            take_min = jnp.logical_not(upper) == ((lane8 & k) == 0)
            take_self = self_less == take_min
            P = jnp.where(take_self, P, pP)
            S = jnp.where(take_self, S, pS)
            d //= 2
        k *= 2

    for b in range(_B):
        perm = S[b:b + 1, :]
        pm = (io0 == perm).astype(f32)
        gb11 = jnp.concatenate([g9_list[b], moc_list[b], gac_list[b]],
                               axis=0)
        fin = jax.lax.dot_general(gb11, pm, _DNL,
                                  precision=_HIGHEST,
                                  preferred_element_type=f32)
        g_ref[b] = fin
        oh = (giota_f == fin[10:11]).astype(f32)
        gtout_ref[b] = jax.lax.dot_general(gtt_ref[b], oh, _DNL,
                                           precision=_HIGHEST,
                                           preferred_element_type=f32)
        mo_s = fin[9:10]
        reg_ref[b] = (mo_s > _REG_FG_THRESH).astype(i32)
        cls_ref[b] = jnp.clip((mo_s - _CLS_BG_THRESH)
                              / (_CLS_FG_THRESH - _CLS_BG_THRESH), 0.0, 1.0)


@jax.jit
def _run(data, gtb, gtt):
    f32 = jnp.float32
    i32 = jnp.int32
    out_shapes = [
        jax.ShapeDtypeStruct((_B, 11, 128), f32),
        jax.ShapeDtypeStruct((_B, 8, 128), f32),
        jax.ShapeDtypeStruct((_B, 1, 128), i32),
        jax.ShapeDtypeStruct((_B, 1, 128), f32),
    ]
    return pl.pallas_call(_tc_body, out_shape=out_shapes)(data, gtb, gtt)


def kernel(rois, roi_scores, gt_boxes, roi_labels, batch_size):
    del batch_size
    f32 = jnp.float32
    roisT = jnp.transpose(rois, (0, 2, 1))
    x, y, z = roisT[:, 0:1], roisT[:, 1:2], roisT[:, 2:3]
    dx, dy, dz = roisT[:, 3:4], roisT[:, 4:5], roisT[:, 5:6]
    bounds = jnp.concatenate(
        [x + dx * 0.5, x - dx * 0.5, y + dy * 0.5, y - dy * 0.5,
         z + dz * 0.5, z - dz * 0.5, dx * dy * dz], axis=1)
    data = jnp.concatenate(
        [roisT, roi_scores[:, None, :], roi_labels.astype(f32)[:, None, :],
         bounds], axis=1)
    data = jnp.pad(data, ((0, 0), (0, 0), (0, _NRP - _NR)))
    gpad = jnp.tile(
        jnp.array([1e9, 1e9, 1e9, 1.0, 1.0, 1.0, 0.0, 0.0], f32)[None, None],
        (_B, _NGP - _NG, 1))
    gtc = jnp.concatenate([gt_boxes, gpad], axis=1)
    gx, gy, gz = gtc[:, :, 0:1], gtc[:, :, 1:2], gtc[:, :, 2:3]
    gdx, gdy, gdz = gtc[:, :, 3:4], gtc[:, :, 4:5], gtc[:, :, 5:6]
    gtb = jnp.concatenate(
        [gx + gdx * 0.5, gx - gdx * 0.5, gy + gdy * 0.5, gy - gdy * 0.5,
         gz + gdz * 0.5, gz - gdz * 0.5, gdx * gdy * gdz,
         jnp.zeros_like(gx)], axis=2)
    gtt = jnp.transpose(gtc, (0, 2, 1))

    g, gtout, reg, cls = _run(data, gtb, gtt)

    batch_rois = jnp.transpose(g[:, 0:7, :], (0, 2, 1))
    batch_gt_of_rois = jnp.transpose(gtout, (0, 2, 1))
    batch_roi_ious = g[:, 9, :]
    batch_roi_scores = g[:, 7, :]
    batch_roi_labels = g[:, 8, :].astype(roi_labels.dtype)
    reg_valid_mask = reg[:, 0, :]
    batch_cls_labels = cls[:, 0, :]
    return (batch_rois, batch_gt_of_rois, batch_roi_ious, batch_roi_scores,
            batch_roi_labels, reg_valid_mask, batch_cls_labels)


# speedup vs baseline: 1.0816x; 1.0816x over previous
import jax
import jax.numpy as jnp
from jax.experimental import pallas as pl
from jax.experimental.pallas import tpu as pltpu

_B = 8
_NR = 5000
_NRP = 5120
_NCH = _NRP // 128
_NG = 100
_NGP = 104
_FG = 64
_REG_FG_THRESH = 0.55
_CLS_FG_THRESH = 0.75
_CLS_BG_THRESH = 0.25

_HIGHEST = jax.lax.Precision.HIGHEST
_DNL = (((1,), (0,)), ((), ()))
_DNT = (((1,), (1,)), ((), ()))


def _roll(x, s):
    s = s % x.shape[1]
    if s == 0:
        return x
    return jnp.concatenate([x[:, -s:], x[:, :-s]], axis=1)


def _iou_block(axM, axm, ayM, aym, azM, azm, va, gb):
    xM, xm = gb[:, 0:1], gb[:, 1:2]
    yM, ym = gb[:, 2:3], gb[:, 3:4]
    zM, zm = gb[:, 4:5], gb[:, 5:6]
    vb = gb[:, 6:7]
    ox = jnp.maximum(jnp.minimum(axM, xM) - jnp.maximum(axm, xm), 0.0)
    oy = jnp.maximum(jnp.minimum(ayM, yM) - jnp.maximum(aym, ym), 0.0)
    oz = jnp.maximum(jnp.minimum(azM, zM) - jnp.maximum(azm, zm), 0.0)
    inter = ox * oy * oz
    return inter / jnp.maximum((va + vb) - inter, 1e-6)


def _tc_body(data_ref, gtb_ref, gtt_ref, g_ref, gtout_ref, reg_ref, cls_ref):
    f32 = jnp.float32
    i32 = jnp.int32

    mo_imgs = []
    for b in range(_B):
        d = data_ref[b]
        gb = gtb_ref[b]
        rows = []
        for j in range(_NCH):
            lo = j * 128
            iou = _iou_block(
                d[9:10, lo:lo + 128], d[10:11, lo:lo + 128],
                d[11:12, lo:lo + 128], d[12:13, lo:lo + 128],
                d[13:14, lo:lo + 128], d[14:15, lo:lo + 128],
                d[15:16, lo:lo + 128], gb)
            rows.append(jnp.max(iou, axis=0, keepdims=True))
        mo_imgs.append(jnp.concatenate(rows, axis=0))
    mo3 = jnp.stack(mo_imgs)

    r3 = jax.lax.broadcasted_iota(i32, (_B, _NCH, 128), 1)
    c3 = jax.lax.broadcasted_iota(i32, (_B, _NCH, 128), 2)
    valid3 = (r3 * 128 + c3) < _NR
    bits3 = jax.lax.bitcast_convert_type(mo3, i32)
    bitsf = jnp.where(valid3, bits3, -1)
    bitsb = jnp.where(valid3, bits3, jnp.int32(0x7FFFFFFF))

    tfb_i = jnp.zeros((_B, 1, 1), i32)
    tbb_i = jnp.full((_B, 1, 1), 0x7FFFFFFF, i32)
    for i in range(31):
        bit = jnp.int32(1 << (30 - i))
        candf = tfb_i | bit
        cf = jnp.sum((bitsf >= candf).astype(i32), axis=(1, 2), keepdims=True)
        tfb_i = jnp.where(cf >= _FG, candf, tfb_i)
        candb = tbb_i & jnp.bitwise_not(bit)
        cb = jnp.sum((bitsb <= candb).astype(i32), axis=(1, 2), keepdims=True)
        tbb_i = jnp.where(cb >= _FG, candb, tbb_i)
    tf3 = jax.lax.bitcast_convert_type(tfb_i, f32)
    tb3 = jax.lax.bitcast_convert_type(tbb_i, f32)

    io0 = jax.lax.broadcasted_iota(i32, (128, 128), 0)
    io1 = jax.lax.broadcasted_iota(i32, (128, 128), 1)
    tl128 = (io0 <= io1).astype(f32)
    i128 = (io0 == io1).astype(f32)
    q0 = jax.lax.broadcasted_iota(i32, (4 * _NCH, 4 * _NCH), 0)
    q1 = jax.lax.broadcasted_iota(i32, (4 * _NCH, 4 * _NCH), 1)
    same = (((q0 < 40) & (q1 < 40))
            | ((q0 >= 40) & (q0 < 80) & (q1 >= 40) & (q1 < 80))
            | ((q0 >= 80) & (q0 < 120) & (q1 >= 80) & (q1 < 120))
            | ((q0 >= 120) & (q1 >= 120)))
    sblk = (same & (q1 < q0)).astype(f32)
    r2d = jax.lax.broadcasted_iota(i32, (_NCH, 128), 0)
    c2d = jax.lax.broadcasted_iota(i32, (_NCH, 128), 1)
    valid2 = (r2d * 128 + c2d) < _NR
    lane_f = jax.lax.broadcasted_iota(i32, (1, 128), 1).astype(f32)
    giota = jax.lax.broadcasted_iota(i32, (_NGP, 1), 0)
    giota_f = giota.astype(f32)

    moc_list = []
    gac_list = []
    g9_list = []
    for b in range(_B):
        mob = mo_imgs[b]
        tfv = tf3[b]
        tbv = tb3[b]
        betf = valid2 & (mob > tfv)
        tief = valid2 & (mob == tfv)
        betb = valid2 & (mob < tbv)
        tieb = valid2 & (mob == tbv)
        needf = jnp.float32(_FG) - jnp.sum(betf.astype(f32), keepdims=True)
        needb = jnp.float32(_FG) - jnp.sum(betb.astype(f32), keepdims=True)
        mstk = jnp.concatenate([betf, tief, betb, tieb],
                               axis=0).astype(f32)
        inc = jax.lax.dot_general(mstk, tl128, _DNL,
                                  preferred_element_type=f32)
        rowoff = jax.lax.dot_general(sblk, inc[:, 127:128], _DNL,
                                     preferred_element_type=f32)
        pstk = (inc - mstk) + rowoff
        pbf, ptf = pstk[0:40], pstk[40:80]
        pbb, ptb = pstk[80:120], pstk[120:160]
        self_f = betf | (tief & (ptf < needf))
        sf = jnp.where(self_f, pbf + jnp.minimum(ptf, needf), 200.0)
        self_b = betb | (tieb & (ptb < needb))
        sb_ = jnp.where(self_b, 64.0 + pbb + jnp.minimum(ptb, needb), 200.0)
        s2 = jnp.concatenate([sf, sb_], axis=0)
        scol = jax.lax.dot_general(i128, s2, _DNT,
                                   preferred_element_type=f32)
        blocks = []
        for r in range(_NCH):
            blk = ((scol[:, r:r + 1] == lane_f)
                   | (scol[:, _NCH + r:_NCH + r + 1] == lane_f))
            blocks.append(blk.astype(f32))
        sel = jnp.concatenate(blocks, axis=0)
        g9 = jax.lax.dot_general(data_ref[b][0:9], sel, _DNL,
                                 precision=_HIGHEST,
                                 preferred_element_type=f32)
        x, y, z = g9[0:1], g9[1:2], g9[2:3]
        dx, dy, dz = g9[3:4], g9[4:5], g9[5:6]
        iou2 = _iou_block(x + dx * 0.5, x - dx * 0.5,
                          y + dy * 0.5, y - dy * 0.5,
                          z + dz * 0.5, z - dz * 0.5,
                          (dx * dy) * dz, gtb_ref[b])
        moc = jnp.max(iou2, axis=0, keepdims=True)
        gac = jnp.min(jnp.where(iou2 == moc, giota, _NGP),
                      axis=0, keepdims=True)
        moc_list.append(moc)
        gac_list.append(gac.astype(f32))
        g9_list.append(g9)

    V = jnp.concatenate(moc_list, axis=0)
    bitsV = jax.lax.bitcast_convert_type(V, i32)
    lane8 = jax.lax.broadcasted_iota(i32, (_B, 128), 1)
    half_fg = lane8 < _FG
    P = jnp.where(half_fg, -bitsV, bitsV + jnp.int32(1 << 30))
    S = lane8
    k = 2
    while k <= 128:
        d = k // 2
        while d >= 1:
            upper = (lane8 & d) != 0
            pP = jnp.where(upper, _roll(P, d), _roll(P, -d))
            pS = jnp.where(upper, _roll(S, d), _roll(S, -d))
            self_less = (P < pP) | ((P == pP) & (S < pS))
            take_min = jnp.logical_not(upper) == ((lane8 & k) == 0)
            take_self = self_less == take_min
            P = jnp.where(take_self, P, pP)
            S = jnp.where(take_self, S, pS)
            d //= 2
        k *= 2

    for b in range(_B):
        perm = S[b:b + 1, :]
        pm = (io0 == perm).astype(f32)
        gb11 = jnp.concatenate([g9_list[b], moc_list[b], gac_list[b]],
                               axis=0)
        fin = jax.lax.dot_general(gb11, pm, _DNL,
                                  precision=_HIGHEST,
                                  preferred_element_type=f32)
        g_ref[b] = fin
        oh = (giota_f == fin[10:11]).astype(f32)
        gtout_ref[b] = jax.lax.dot_general(gtt_ref[b], oh, _DNL,
                                           precision=_HIGHEST,
                                           preferred_element_type=f32)
        mo_s = fin[9:10]
        reg_ref[b] = (mo_s > _REG_FG_THRESH).astype(i32)
        cls_ref[b] = jnp.clip((mo_s - _CLS_BG_THRESH)
                              / (_CLS_FG_THRESH - _CLS_BG_THRESH), 0.0, 1.0)


@jax.jit
def _run(data, gtb, gtt):
    f32 = jnp.float32
    i32 = jnp.int32
    out_shapes = [
        jax.ShapeDtypeStruct((_B, 11, 128), f32),
        jax.ShapeDtypeStruct((_B, 8, 128), f32),
        jax.ShapeDtypeStruct((_B, 1, 128), i32),
        jax.ShapeDtypeStruct((_B, 1, 128), f32),
    ]
    return pl.pallas_call(_tc_body, out_shape=out_shapes)(data, gtb, gtt)


def kernel(rois, roi_scores, gt_boxes, roi_labels, batch_size):
    del batch_size
    f32 = jnp.float32
    roisT = jnp.transpose(rois, (0, 2, 1))
    x, y, z = roisT[:, 0:1], roisT[:, 1:2], roisT[:, 2:3]
    dx, dy, dz = roisT[:, 3:4], roisT[:, 4:5], roisT[:, 5:6]
    bounds = jnp.concatenate(
        [x + dx * 0.5, x - dx * 0.5, y + dy * 0.5, y - dy * 0.5,
         z + dz * 0.5, z - dz * 0.5, dx * dy * dz], axis=1)
    data = jnp.concatenate(
        [roisT, roi_scores[:, None, :], roi_labels.astype(f32)[:, None, :],
         bounds], axis=1)
    data = jnp.pad(data, ((0, 0), (0, 0), (0, _NRP - _NR)))
    gpad = jnp.tile(
        jnp.array([1e9, 1e9, 1e9, 1.0, 1.0, 1.0, 0.0, 0.0], f32)[None, None],
        (_B, _NGP - _NG, 1))
    gtc = jnp.concatenate([gt_boxes, gpad], axis=1)
    gx, gy, gz = gtc[:, :, 0:1], gtc[:, :, 1:2], gtc[:, :, 2:3]
    gdx, gdy, gdz = gtc[:, :, 3:4], gtc[:, :, 4:5], gtc[:, :, 5:6]
    gtb = jnp.concatenate(
        [gx + gdx * 0.5, gx - gdx * 0.5, gy + gdy * 0.5, gy - gdy * 0.5,
         gz + gdz * 0.5, gz - gdz * 0.5, gdx * gdy * gdz,
         jnp.zeros_like(gx)], axis=2)
    gtt = jnp.transpose(gtc, (0, 2, 1))

    g, gtout, reg, cls = _run(data, gtb, gtt)

    batch_rois = jnp.transpose(g[:, 0:7, :], (0, 2, 1))
    batch_gt_of_rois = jnp.transpose(gtout, (0, 2, 1))
    batch_roi_ious = g[:, 9, :]
    batch_roi_scores = g[:, 7, :]
    batch_roi_labels = g[:, 8, :].astype(roi_labels.dtype)
    reg_valid_mask = reg[:, 0, :]
    batch_cls_labels = cls[:, 0, :]
    return (batch_rois, batch_gt_of_rois, batch_roi_ious, batch_roi_scores,
            batch_roi_labels, reg_valid_mask, batch_cls_labels)


# speedup vs baseline: 1.0924x; 1.0101x over previous
import jax
import jax.numpy as jnp
from jax.experimental import pallas as pl
from jax.experimental.pallas import tpu as pltpu

_B = 8
_NR = 5000
_NRP = 5120
_NCH = _NRP // 128
_NG = 100
_NGP = 104
_FG = 64
_REG_FG_THRESH = 0.55
_CLS_FG_THRESH = 0.75
_CLS_BG_THRESH = 0.25

_HIGHEST = jax.lax.Precision.HIGHEST
_DNL = (((1,), (0,)), ((), ()))
_DNT = (((1,), (1,)), ((), ()))


def _roll(x, s):
    s = s % x.shape[1]
    if s == 0:
        return x
    return jnp.concatenate([x[:, -s:], x[:, :-s]], axis=1)


def _iou_block(axM, axm, ayM, aym, azM, azm, va, gb):
    xM, xm = gb[:, 0:1], gb[:, 1:2]
    yM, ym = gb[:, 2:3], gb[:, 3:4]
    zM, zm = gb[:, 4:5], gb[:, 5:6]
    vb = gb[:, 6:7]
    ox = jnp.maximum(jnp.minimum(axM, xM) - jnp.maximum(axm, xm), 0.0)
    oy = jnp.maximum(jnp.minimum(ayM, yM) - jnp.maximum(aym, ym), 0.0)
    oz = jnp.maximum(jnp.minimum(azM, zM) - jnp.maximum(azm, zm), 0.0)
    inter = ox * oy * oz
    return inter / jnp.maximum((va + vb) - inter, 1e-6)


def _tc_body(data_ref, gtb_ref, gtt_ref, g_ref, gtout_ref, reg_ref, cls_ref):
    f32 = jnp.float32
    i32 = jnp.int32

    mo_imgs = []
    for b in range(_B):
        d = data_ref[b]
        gb = gtb_ref[b]
        rows = []
        for j in range(_NCH):
            lo = j * 128
            axM = d[9:10, lo:lo + 128]
            axm = d[10:11, lo:lo + 128]
            ayM = d[11:12, lo:lo + 128]
            aym = d[12:13, lo:lo + 128]
            azM = d[13:14, lo:lo + 128]
            azm = d[14:15, lo:lo + 128]
            va = d[15:16, lo:lo + 128]
            macc = None
            for g in range(_NGP // 8):
                g8 = g * 8
                sub = gb[g8:g8 + 8]
                ox = jnp.maximum(
                    jnp.minimum(axM, sub[:, 0:1])
                    - jnp.maximum(axm, sub[:, 1:2]), 0.0)
                oy = jnp.maximum(
                    jnp.minimum(ayM, sub[:, 2:3])
                    - jnp.maximum(aym, sub[:, 3:4]), 0.0)
                oz = jnp.maximum(
                    jnp.minimum(azM, sub[:, 4:5])
                    - jnp.maximum(azm, sub[:, 5:6]), 0.0)
                inter = ox * oy * oz
                iou = inter / jnp.maximum((va + sub[:, 6:7]) - inter, 1e-6)
                macc = iou if macc is None else jnp.maximum(macc, iou)
            rows.append(jnp.max(macc, axis=0, keepdims=True))
        mo_imgs.append(jnp.concatenate(rows, axis=0))
    mo3 = jnp.stack(mo_imgs)

    r3 = jax.lax.broadcasted_iota(i32, (_B, _NCH, 128), 1)
    c3 = jax.lax.broadcasted_iota(i32, (_B, _NCH, 128), 2)
    valid3 = (r3 * 128 + c3) < _NR
    bits3 = jax.lax.bitcast_convert_type(mo3, i32)
    bitsf = jnp.where(valid3, bits3, -1)
    bitsb = jnp.where(valid3, bits3, jnp.int32(0x7FFFFFFF))

    tfb_i = jnp.zeros((_B, 1, 1), i32)
    tbb_i = jnp.full((_B, 1, 1), 0x7FFFFFFF, i32)
    for i in range(31):
        bit = jnp.int32(1 << (30 - i))
        candf = tfb_i | bit
        cf = jnp.sum((bitsf >= candf).astype(i32), axis=(1, 2), keepdims=True)
        tfb_i = jnp.where(cf >= _FG, candf, tfb_i)
        candb = tbb_i & jnp.bitwise_not(bit)
        cb = jnp.sum((bitsb <= candb).astype(i32), axis=(1, 2), keepdims=True)
        tbb_i = jnp.where(cb >= _FG, candb, tbb_i)
    tf3 = jax.lax.bitcast_convert_type(tfb_i, f32)
    tb3 = jax.lax.bitcast_convert_type(tbb_i, f32)

    io0 = jax.lax.broadcasted_iota(i32, (128, 128), 0)
    io1 = jax.lax.broadcasted_iota(i32, (128, 128), 1)
    tl128 = (io0 <= io1).astype(f32)
    i128 = (io0 == io1).astype(f32)
    q0 = jax.lax.broadcasted_iota(i32, (4 * _NCH, 4 * _NCH), 0)
    q1 = jax.lax.broadcasted_iota(i32, (4 * _NCH, 4 * _NCH), 1)
    same = (((q0 < 40) & (q1 < 40))
            | ((q0 >= 40) & (q0 < 80) & (q1 >= 40) & (q1 < 80))
            | ((q0 >= 80) & (q0 < 120) & (q1 >= 80) & (q1 < 120))
            | ((q0 >= 120) & (q1 >= 120)))
    sblk = (same & (q1 < q0)).astype(f32)
    r2d = jax.lax.broadcasted_iota(i32, (_NCH, 128), 0)
    c2d = jax.lax.broadcasted_iota(i32, (_NCH, 128), 1)
    valid2 = (r2d * 128 + c2d) < _NR
    lane_f = jax.lax.broadcasted_iota(i32, (1, 128), 1).astype(f32)
    giota = jax.lax.broadcasted_iota(i32, (_NGP, 1), 0)
    giota_f = giota.astype(f32)

    moc_list = []
    gac_list = []
    g9_list = []
    for b in range(_B):
        mob = mo_imgs[b]
        tfv = tf3[b]
        tbv = tb3[b]
        betf = valid2 & (mob > tfv)
        tief = valid2 & (mob == tfv)
        betb = valid2 & (mob < tbv)
        tieb = valid2 & (mob == tbv)
        needf = jnp.float32(_FG) - jnp.sum(betf.astype(f32), keepdims=True)
        needb = jnp.float32(_FG) - jnp.sum(betb.astype(f32), keepdims=True)
        mstk = jnp.concatenate([betf, tief, betb, tieb],
                               axis=0).astype(f32)
        inc = jax.lax.dot_general(mstk, tl128, _DNL,
                                  preferred_element_type=f32)
        rowoff = jax.lax.dot_general(sblk, inc[:, 127:128], _DNL,
                                     preferred_element_type=f32)
        pstk = (inc - mstk) + rowoff
        pbf, ptf = pstk[0:40], pstk[40:80]
        pbb, ptb = pstk[80:120], pstk[120:160]
        self_f = betf | (tief & (ptf < needf))
        sf = jnp.where(self_f, pbf + jnp.minimum(ptf, needf), 200.0)
        self_b = betb | (tieb & (ptb < needb))
        sb_ = jnp.where(self_b, 64.0 + pbb + jnp.minimum(ptb, needb), 200.0)
        s2 = jnp.concatenate([sf, sb_], axis=0)
        scol = jax.lax.dot_general(i128, s2, _DNT,
                                   preferred_element_type=f32)
        blocks = []
        for r in range(_NCH):
            blk = ((scol[:, r:r + 1] == lane_f)
                   | (scol[:, _NCH + r:_NCH + r + 1] == lane_f))
            blocks.append(blk.astype(f32))
        sel = jnp.concatenate(blocks, axis=0)
        g9 = jax.lax.dot_general(data_ref[b][0:9], sel, _DNL,
                                 precision=_HIGHEST,
                                 preferred_element_type=f32)
        x, y, z = g9[0:1], g9[1:2], g9[2:3]
        dx, dy, dz = g9[3:4], g9[4:5], g9[5:6]
        iou2 = _iou_block(x + dx * 0.5, x - dx * 0.5,
                          y + dy * 0.5, y - dy * 0.5,
                          z + dz * 0.5, z - dz * 0.5,
                          (dx * dy) * dz, gtb_ref[b])
        moc = jnp.max(iou2, axis=0, keepdims=True)
        gac = jnp.min(jnp.where(iou2 == moc, giota, _NGP),
                      axis=0, keepdims=True)
        moc_list.append(moc)
        gac_list.append(gac.astype(f32))
        g9_list.append(g9)

    V = jnp.concatenate(moc_list, axis=0)
    bitsV = jax.lax.bitcast_convert_type(V, i32)
    lane8 = jax.lax.broadcasted_iota(i32, (_B, 128), 1)
    half_fg = lane8 < _FG
    P = jnp.where(half_fg, -bitsV, bitsV + jnp.int32(1 << 30))
    S = lane8
    k = 2
    while k <= 128:
        d = k // 2
        while d >= 1:
            upper = (lane8 & d) != 0
            pP = jnp.where(upper, _roll(P, d), _roll(P, -d))
            pS = jnp.where(upper, _roll(S, d), _roll(S, -d))
            self_less = (P < pP) | ((P == pP) & (S < pS))
            take_min = jnp.logical_not(upper) == ((lane8 & k) == 0)
            take_self = self_less == take_min
            P = jnp.where(take_self, P, pP)
            S = jnp.where(take_self, S, pS)
            d //= 2
        k *= 2

    for b in range(_B):
        perm = S[b:b + 1, :]
        pm = (io0 == perm).astype(f32)
        gb11 = jnp.concatenate([g9_list[b], moc_list[b], gac_list[b]],
                               axis=0)
        fin = jax.lax.dot_general(gb11, pm, _DNL,
                                  precision=_HIGHEST,
                                  preferred_element_type=f32)
        g_ref[b] = fin
        oh = (giota_f == fin[10:11]).astype(f32)
        gtout_ref[b] = jax.lax.dot_general(gtt_ref[b], oh, _DNL,
                                           precision=_HIGHEST,
                                           preferred_element_type=f32)
        mo_s = fin[9:10]
        reg_ref[b] = (mo_s > _REG_FG_THRESH).astype(i32)
        cls_ref[b] = jnp.clip((mo_s - _CLS_BG_THRESH)
                              / (_CLS_FG_THRESH - _CLS_BG_THRESH), 0.0, 1.0)


@jax.jit
def _run(data, gtb, gtt):
    f32 = jnp.float32
    i32 = jnp.int32
    out_shapes = [
        jax.ShapeDtypeStruct((_B, 11, 128), f32),
        jax.ShapeDtypeStruct((_B, 8, 128), f32),
        jax.ShapeDtypeStruct((_B, 1, 128), i32),
        jax.ShapeDtypeStruct((_B, 1, 128), f32),
    ]
    return pl.pallas_call(_tc_body, out_shape=out_shapes)(data, gtb, gtt)


def kernel(rois, roi_scores, gt_boxes, roi_labels, batch_size):
    del batch_size
    f32 = jnp.float32
    roisT = jnp.transpose(rois, (0, 2, 1))
    x, y, z = roisT[:, 0:1], roisT[:, 1:2], roisT[:, 2:3]
    dx, dy, dz = roisT[:, 3:4], roisT[:, 4:5], roisT[:, 5:6]
    bounds = jnp.concatenate(
        [x + dx * 0.5, x - dx * 0.5, y + dy * 0.5, y - dy * 0.5,
         z + dz * 0.5, z - dz * 0.5, dx * dy * dz], axis=1)
    data = jnp.concatenate(
        [roisT, roi_scores[:, None, :], roi_labels.astype(f32)[:, None, :],
         bounds], axis=1)
    data = jnp.pad(data, ((0, 0), (0, 0), (0, _NRP - _NR)))
    gpad = jnp.tile(
        jnp.array([1e9, 1e9, 1e9, 1.0, 1.0, 1.0, 0.0, 0.0], f32)[None, None],
        (_B, _NGP - _NG, 1))
    gtc = jnp.concatenate([gt_boxes, gpad], axis=1)
    gx, gy, gz = gtc[:, :, 0:1], gtc[:, :, 1:2], gtc[:, :, 2:3]
    gdx, gdy, gdz = gtc[:, :, 3:4], gtc[:, :, 4:5], gtc[:, :, 5:6]
    gtb = jnp.concatenate(
        [gx + gdx * 0.5, gx - gdx * 0.5, gy + gdy * 0.5, gy - gdy * 0.5,
         gz + gdz * 0.5, gz - gdz * 0.5, gdx * gdy * gdz,
         jnp.zeros_like(gx)], axis=2)
    gtt = jnp.transpose(gtc, (0, 2, 1))

    g, gtout, reg, cls = _run(data, gtb, gtt)

    batch_rois = jnp.transpose(g[:, 0:7, :], (0, 2, 1))
    batch_gt_of_rois = jnp.transpose(gtout, (0, 2, 1))
    batch_roi_ious = g[:, 9, :]
    batch_roi_scores = g[:, 7, :]
    batch_roi_labels = g[:, 8, :].astype(roi_labels.dtype)
    reg_valid_mask = reg[:, 0, :]
    batch_cls_labels = cls[:, 0, :]
    return (batch_rois, batch_gt_of_rois, batch_roi_ious, batch_roi_scores,
            batch_roi_labels, reg_valid_mask, batch_cls_labels)
